# baseline (speedup 1.0000x reference)
import jax
import jax.numpy as jnp
from jax import lax
from jax.experimental import pallas as pl
from jax.experimental.pallas import tpu as pltpu

_N = 1000
_D = 1024
_R = 16
_K = 50
_BETA2 = 0.2 ** 2
_TAU = 0.01
_SCALING = 2.0
_KPAD = 56
_CCH = 128
_NCH = _D // _CCH


def _body(q_ref, cor_hbm, wb_hbm, a_hbm, bt_hbm, out_ref,
          idx_sm, idx_v, acat, bcat, cbuf, wbuf,
          a_sem, b_sem, i_sem, c_sem, w_sem):
    def c_copy(t):
        return pltpu.make_async_copy(
            cor_hbm.at[:, pl.ds(t * _CCH, _CCH)], cbuf.at[t % 2],
            c_sem.at[t % 2])

    w_copy = pltpu.make_async_copy(wb_hbm, wbuf, w_sem)
    w_copy.start()
    c_copy(0).start()
    c_copy(1).start()

    q = q_ref[...]
    qn = jnp.sqrt(jnp.sum(q * q))

    ii = lax.broadcasted_iota(jnp.int32, (_N, _N), 0)
    jj = lax.broadcasted_iota(jnp.int32, (_N, _N), 1)
    eye_n = jnp.where(ii == jj, 1.0, 0.0)
    rowg = lax.broadcasted_iota(jnp.int32, (_KPAD * _R, 64), 0) // _R
    kcol = lax.broadcasted_iota(jnp.int32, (_KPAD * _R, 64), 1)
    eye_g = jnp.where(rowg == kcol, 1.0, 0.0)

    scores = jnp.zeros((1, _N), jnp.float32)
    csq = jnp.zeros((1, _N), jnp.float32)
    ones = jnp.ones((1, _CCH), jnp.float32)
    for t in range(_NCH):
        c_copy(t).wait()
        ch = cbuf[t % 2]
        qc = q[:, t * _CCH:(t + 1) * _CCH]
        scores = scores + lax.dot_general(
            qc, ch, (((1,), (1,)), ((), ())),
            preferred_element_type=jnp.float32)
        csq = csq + lax.dot_general(
            ones, ch * ch, (((1,), (1,)), ((), ())),
            preferred_element_type=jnp.float32)
        if t + 2 < _NCH:
            c_copy(t + 2).start()

    cn = jnp.sqrt(csq)
    sim = scores / ((cn + 1e-9) * (qn + 1e-9)) / _BETA2
    mx = jnp.max(sim)
    e = jnp.exp(sim - mx)
    p = e / jnp.sum(e)
    p = jnp.where(p >= _TAU, p, 0.0)

    p_col = lax.dot_general(eye_n, p, (((1,), (1,)), ((), ())),
                            preferred_element_type=jnp.float32)
    cmp = jnp.where(p_col < p, 1.0, 0.0)
    rank = lax.dot_general(cmp, jnp.ones((_N, 1), jnp.float32),
                           (((1,), (0,)), ((), ())),
                           preferred_element_type=jnp.float32)
    lane64 = lax.broadcasted_iota(jnp.int32, (1, 64), 1).astype(jnp.float32)
    onehot = jnp.where(rank == lane64, 1.0, 0.0)
    lane_f = lax.broadcasted_iota(jnp.int32, (1, _N), 1).astype(jnp.float32)
    idx_f = lax.dot_general(lane_f, onehot, (((1,), (0,)), ((), ())),
                            preferred_element_type=jnp.float32)
    wvec = lax.dot_general(p, onehot, (((1,), (0,)), ((), ())),
                           preferred_element_type=jnp.float32)
    sel = lane64 < float(_K)
    ssum = jnp.sum(jnp.where(sel, wvec, 0.0))
    wscale = _SCALING / (ssum + 1e-9)
    scale64 = jnp.where(sel, wvec * wscale, 0.0)
    idx_v[...] = jnp.clip(idx_f, 0.0, float(_N - 1)).astype(jnp.int32)
    pltpu.make_async_copy(idx_v, idx_sm, i_sem).start()
    pltpu.make_async_copy(idx_v, idx_sm, i_sem).wait()

    def a_copy(k):
        return pltpu.make_async_copy(
            a_hbm.at[idx_sm[0, k]], acat.at[pl.ds(k * _R, _R), :], a_sem)

    def b_copy(k):
        return pltpu.make_async_copy(
            bt_hbm.at[idx_sm[0, k]], bcat.at[pl.ds(k * _R, _R), :], b_sem)

    for k in range(_K):
        a_copy(k).start()
        b_copy(k).start()
    zpad = jnp.zeros(((_KPAD - _K) * _R, _D), jnp.float32)
    bcat[pl.ds(_K * _R, (_KPAD - _K) * _R), :] = zpad
    acat[pl.ds(_K * _R, (_KPAD - _K) * _R), :] = zpad

    scale_col = lax.dot_general(eye_g, scale64, (((1,), (1,)), ((), ())),
                                preferred_element_type=jnp.float32)

    for k in range(_K):
        b_copy(k).wait()
    for k in range(_K):
        a_copy(k).wait()

    delta = lax.dot_general(
        (bcat[...] * scale_col).astype(jnp.bfloat16),
        acat[...].astype(jnp.bfloat16),
        (((0,), (0,)), ((), ())), preferred_element_type=jnp.float32)
    w_copy.wait()
    out_ref[...] = wbuf[...] + delta


def kernel(q, corpus, A_all, B_all, W_base):
    B_t = jnp.swapaxes(B_all, 1, 2)
    return pl.pallas_call(
        _body,
        out_shape=jax.ShapeDtypeStruct((_D, _D), jnp.float32),
        in_specs=[
            pl.BlockSpec(memory_space=pltpu.VMEM),
            pl.BlockSpec(memory_space=pltpu.HBM),
            pl.BlockSpec(memory_space=pltpu.HBM),
            pl.BlockSpec(memory_space=pltpu.HBM),
            pl.BlockSpec(memory_space=pltpu.HBM),
        ],
        out_specs=pl.BlockSpec(memory_space=pltpu.VMEM),
        scratch_shapes=[
            pltpu.SMEM((1, 64), jnp.int32),
            pltpu.VMEM((1, 64), jnp.int32),
            pltpu.VMEM((_KPAD * _R, _D), jnp.float32),
            pltpu.VMEM((_KPAD * _R, _D), jnp.float32),
            pltpu.VMEM((2, _N, _CCH), jnp.float32),
            pltpu.VMEM((_D, _D), jnp.float32),
            pltpu.SemaphoreType.DMA,
            pltpu.SemaphoreType.DMA,
            pltpu.SemaphoreType.DMA,
            pltpu.SemaphoreType.DMA((2,)),
            pltpu.SemaphoreType.DMA,
        ],
    )(q, corpus, W_base, A_all, B_t)


# speedup vs baseline: 1.1900x; 1.1900x over previous
import jax
import jax.numpy as jnp
from jax import lax
from jax.experimental import pallas as pl
from jax.experimental.pallas import tpu as pltpu

_N = 1000
_D = 1024
_R = 16
_K = 50
_BETA2 = 0.2 ** 2
_TAU = 0.01
_SCALING = 2.0
_KPAD = 56
_CCH = 128
_NCH = _D // _CCH


def _body(q_ref, cor_ref, wb_hbm, a_hbm, bt_hbm, out_ref,
          idx_sm, idx_v, acat, bcat, wbuf,
          a_sem, b_sem, i_sem, w_sem):
    w_copy = pltpu.make_async_copy(wb_hbm, wbuf, w_sem)
    w_copy.start()

    q = q_ref[...]
    qn = jnp.sqrt(jnp.sum(q * q))

    ii = lax.broadcasted_iota(jnp.int32, (_N, _N), 0)
    jj = lax.broadcasted_iota(jnp.int32, (_N, _N), 1)
    eye_n = jnp.where(ii == jj, 1.0, 0.0)
    rowg = lax.broadcasted_iota(jnp.int32, (_KPAD * _R, 64), 0) // _R
    kcol = lax.broadcasted_iota(jnp.int32, (_KPAD * _R, 64), 1)
    eye_g = jnp.where(rowg == kcol, 1.0, 0.0)

    scores = lax.dot_general(q, cor_ref[...], (((1,), (1,)), ((), ())),
                             preferred_element_type=jnp.float32)
    csq = jnp.zeros((1, _N), jnp.float32)
    ones = jnp.ones((1, _CCH), jnp.float32)
    for t in range(_NCH):
        ch = cor_ref[:, 128 * t:128 * (t + 1)]
        csq = csq + lax.dot_general(
            ones, ch * ch, (((1,), (1,)), ((), ())),
            preferred_element_type=jnp.float32)

    cn = jnp.sqrt(csq)
    sim = scores / ((cn + 1e-9) * (qn + 1e-9)) / _BETA2
    mx = jnp.max(sim)
    e = jnp.exp(sim - mx)
    p = e / jnp.sum(e)
    p = jnp.where(p >= _TAU, p, 0.0)

    p_col = lax.dot_general(eye_n, p, (((1,), (1,)), ((), ())),
                            preferred_element_type=jnp.float32)
    cmp = jnp.where(p_col < p, 1.0, 0.0)
    rank = lax.dot_general(cmp, jnp.ones((_N, 1), jnp.float32),
                           (((1,), (0,)), ((), ())),
                           preferred_element_type=jnp.float32)
    lane64 = lax.broadcasted_iota(jnp.int32, (1, 64), 1).astype(jnp.float32)
    onehot = jnp.where(rank == lane64, 1.0, 0.0)
    lane_f = lax.broadcasted_iota(jnp.int32, (1, _N), 1).astype(jnp.float32)
    idx_f = lax.dot_general(lane_f, onehot, (((1,), (0,)), ((), ())),
                            preferred_element_type=jnp.float32)
    wvec = lax.dot_general(p, onehot, (((1,), (0,)), ((), ())),
                           preferred_element_type=jnp.float32)
    sel = lane64 < float(_K)
    ssum = jnp.sum(jnp.where(sel, wvec, 0.0))
    wscale = _SCALING / (ssum + 1e-9)
    scale64 = jnp.where(sel, wvec * wscale, 0.0)
    idx_v[...] = jnp.clip(idx_f, 0.0, float(_N - 1)).astype(jnp.int32)
    pltpu.make_async_copy(idx_v, idx_sm, i_sem).start()
    pltpu.make_async_copy(idx_v, idx_sm, i_sem).wait()

    def a_copy(k):
        return pltpu.make_async_copy(
            a_hbm.at[idx_sm[0, k]], acat.at[pl.ds(k * _R, _R), :], a_sem)

    def b_copy(k):
        return pltpu.make_async_copy(
            bt_hbm.at[idx_sm[0, k]], bcat.at[pl.ds(k * _R, _R), :], b_sem)

    for k in range(_K):
        a_copy(k).start()
        b_copy(k).start()
    zpad = jnp.zeros(((_KPAD - _K) * _R, _D), jnp.float32)
    bcat[pl.ds(_K * _R, (_KPAD - _K) * _R), :] = zpad
    acat[pl.ds(_K * _R, (_KPAD - _K) * _R), :] = zpad

    scale_col = lax.dot_general(eye_g, scale64, (((1,), (1,)), ((), ())),
                                preferred_element_type=jnp.float32)

    for k in range(_K):
        b_copy(k).wait()
    for k in range(_K):
        a_copy(k).wait()

    delta = lax.dot_general(
        (bcat[...] * scale_col).astype(jnp.bfloat16),
        acat[...].astype(jnp.bfloat16),
        (((0,), (0,)), ((), ())), preferred_element_type=jnp.float32)
    w_copy.wait()
    out_ref[...] = wbuf[...] + delta


def kernel(q, corpus, A_all, B_all, W_base):
    B_t = jnp.swapaxes(B_all, 1, 2)
    return pl.pallas_call(
        _body,
        out_shape=jax.ShapeDtypeStruct((_D, _D), jnp.float32),
        in_specs=[
            pl.BlockSpec(memory_space=pltpu.VMEM),
            pl.BlockSpec(memory_space=pltpu.VMEM),
            pl.BlockSpec(memory_space=pltpu.HBM),
            pl.BlockSpec(memory_space=pltpu.HBM),
            pl.BlockSpec(memory_space=pltpu.HBM),
        ],
        out_specs=pl.BlockSpec(memory_space=pltpu.VMEM),
        scratch_shapes=[
            pltpu.SMEM((1, 64), jnp.int32),
            pltpu.VMEM((1, 64), jnp.int32),
            pltpu.VMEM((_KPAD * _R, _D), jnp.float32),
            pltpu.VMEM((_KPAD * _R, _D), jnp.float32),
            pltpu.VMEM((_D, _D), jnp.float32),
            pltpu.SemaphoreType.DMA,
            pltpu.SemaphoreType.DMA,
            pltpu.SemaphoreType.DMA,
            pltpu.SemaphoreType.DMA,
        ],
    )(q, corpus, W_base, A_all, B_t)
